# baseline (speedup 1.0000x reference)
import jax
import jax.numpy as jnp
from jax.experimental import pallas as pl
from jax.experimental.pallas import tpu as pltpu

_M_RANGE = 2
_K = 5
_N_CLASS = 25
_N_DEPTH = 2
_IM_CH = 3
_EPS = 1e-8
_H = 256
_W = 256
_B = 8


def _shift_y(z, d):
    if d == 0:
        return z
    h, w = z.shape
    zpad = jnp.zeros((abs(d), w), z.dtype)
    if d > 0:
        return jnp.concatenate([z[d:, :], zpad], axis=0)
    return jnp.concatenate([zpad, z[:h + d, :]], axis=0)


def _shift_x(z, d):
    if d == 0:
        return z
    h, w = z.shape
    zpad = jnp.zeros((h, abs(d)), z.dtype)
    if d > 0:
        return jnp.concatenate([z[:, d:], zpad], axis=1)
    return jnp.concatenate([zpad, z[:, :w + d]], axis=1)


def _axis_weights(f_idx, frac):
    lo = 1.0 - frac
    w = []
    prev_hi = None
    for k in range(_K):
        e = f_idx == float(k)
        wk = jnp.where(e, lo, 0.0)
        if prev_hi is not None:
            wk = wk + prev_hi
        w.append(wk)
        if k + 1 < _K:
            prev_hi = jnp.where(e, frac, 0.0)
    return w


def _stencil(mot, dep, im_r, pred, mmask, dmask, seg):
    mx = mot[0]
    my = mot[1]
    fmx = jnp.floor(mx)
    fmy = jnp.floor(my)
    fx = mx - fmx
    fy = my - fmy
    ixf = fmx + float(_M_RANGE)
    iyf = fmy + float(_M_RANGE)

    d1 = (dep[0] == 1).astype(jnp.float32)
    dmask[0] = 1.0 - d1
    dmask[1] = d1

    im = [im_r[ch] for ch in range(_IM_CH)]

    wy = _axis_weights(iyf, fy)
    wx = _axis_weights(ixf, fx)

    sd1 = [_shift_y(d1, r - _M_RANGE) for r in range(_K)]
    sim = [[_shift_y(im[ch], r - _M_RANGE) for r in range(_K)]
           for ch in range(_IM_CH)]
    simd = [[sim[ch][r] * sd1[r] for r in range(_K)]
            for ch in range(_IM_CH)]

    def acc(a, v):
        return v if a is None else a + v

    T = None
    S1 = None
    PT = [None] * _IM_CH
    P1 = [None] * _IM_CH

    for col in range(_K):
        wxc = wx[col]
        aT = None
        aS = None
        aPT = [None] * _IM_CH
        aP1 = [None] * _IM_CH
        for row in range(_K):
            t0 = wy[row] * wxc
            mmask[_K * row + col] = t0
            st0 = _shift_y(t0, row - _M_RANGE)
            aT = acc(aT, st0)
            aS = acc(aS, st0 * sd1[row])
            for ch in range(_IM_CH):
                aPT[ch] = acc(aPT[ch], st0 * sim[ch][row])
                aP1[ch] = acc(aP1[ch], st0 * simd[ch][row])
        dx = col - _M_RANGE
        T = acc(T, _shift_x(aT, dx))
        S1 = acc(S1, _shift_x(aS, dx))
        for ch in range(_IM_CH):
            PT[ch] = acc(PT[ch], _shift_x(aPT[ch], dx))
            P1[ch] = acc(P1[ch], _shift_x(aP1[ch], dx))

    eps_tot = float(_N_CLASS) * _EPS
    total1 = S1 + eps_tot
    total2 = (T - S1) + eps_tot
    ratio1 = 1.0 / total1
    f1 = 1.0 - jnp.maximum(1.0 - ratio1, 0.0)
    sum1 = total1 * f1
    left2 = jnp.maximum(1.0 - sum1, 0.0)
    ratio2 = left2 / total2
    f2 = 1.0 - jnp.maximum(1.0 - ratio2, 0.0)

    f12 = f1 - f2
    for ch in range(_IM_CH):
        pred[ch] = f12 * P1[ch] + f2 * PT[ch]
    seg[0] = 1.0 - (sum1 + total2 * f2)


def _gtnet_kernel(mot_hbm, dep_hbm, im_hbm,
                  pred_hbm, mmask_hbm, dmask_hbm, seg_hbm,
                  mot_buf, dep_buf, im_buf,
                  pred_buf, mmask_buf, dmask_buf, seg_buf,
                  mot_sem, dep_sem, im_sem,
                  pred_sem, mmask_sem, dmask_sem, seg_sem):

    def dma_in(slot, step):
        pltpu.make_async_copy(mot_hbm.at[step], mot_buf.at[slot],
                              mot_sem.at[slot]).start()
        pltpu.make_async_copy(dep_hbm.at[step], dep_buf.at[slot],
                              dep_sem.at[slot]).start()
        pltpu.make_async_copy(im_hbm.at[step, pl.ds(_IM_CH, _IM_CH)],
                              im_buf.at[slot], im_sem.at[slot]).start()

    def wait_in(slot):
        pltpu.make_async_copy(mot_hbm.at[0], mot_buf.at[slot],
                              mot_sem.at[slot]).wait()
        pltpu.make_async_copy(dep_hbm.at[0], dep_buf.at[slot],
                              dep_sem.at[slot]).wait()
        pltpu.make_async_copy(im_hbm.at[0, pl.ds(_IM_CH, _IM_CH)],
                              im_buf.at[slot], im_sem.at[slot]).wait()

    def dma_out(slot, step):
        pltpu.make_async_copy(pred_buf.at[slot], pred_hbm.at[step],
                              pred_sem.at[slot]).start()
        pltpu.make_async_copy(mmask_buf.at[slot], mmask_hbm.at[step],
                              mmask_sem.at[slot]).start()
        pltpu.make_async_copy(dmask_buf.at[slot], dmask_hbm.at[step],
                              dmask_sem.at[slot]).start()
        pltpu.make_async_copy(seg_buf.at[slot], seg_hbm.at[step],
                              seg_sem.at[slot]).start()

    def wait_out(slot):
        pltpu.make_async_copy(pred_buf.at[slot], pred_hbm.at[0],
                              pred_sem.at[slot]).wait()
        pltpu.make_async_copy(mmask_buf.at[slot], mmask_hbm.at[0],
                              mmask_sem.at[slot]).wait()
        pltpu.make_async_copy(dmask_buf.at[slot], dmask_hbm.at[0],
                              dmask_sem.at[slot]).wait()
        pltpu.make_async_copy(seg_buf.at[slot], seg_hbm.at[0],
                              seg_sem.at[slot]).wait()

    dma_in(0, 0)

    def body(step, _):
        cur = jax.lax.rem(step, 2)
        nxt = jax.lax.rem(step + 1, 2)

        @pl.when(step + 1 < _B)
        def _():
            dma_in(nxt, step + 1)

        wait_in(cur)

        @pl.when(step >= 2)
        def _():
            wait_out(cur)

        _stencil(mot_buf.at[cur], dep_buf.at[cur], im_buf.at[cur],
                 pred_buf.at[cur], mmask_buf.at[cur], dmask_buf.at[cur],
                 seg_buf.at[cur])
        dma_out(cur, step)
        return ()

    jax.lax.fori_loop(0, _B, body, ())
    wait_out(jax.lax.rem(_B - 2, 2))
    wait_out(jax.lax.rem(_B - 1, 2))


def kernel(im_input, gt_motion, gt_depth, m_kernel, *, interpret=False):
    B = gt_motion.shape[0]
    dep = gt_depth.astype(jnp.int32)

    out_shape = (
        jax.ShapeDtypeStruct((B, _IM_CH, _H, _W), jnp.float32),
        jax.ShapeDtypeStruct((B, _N_CLASS, _H, _W), jnp.float32),
        jax.ShapeDtypeStruct((B, _N_DEPTH, _H, _W), jnp.float32),
        jax.ShapeDtypeStruct((B, 1, _H, _W), jnp.float32),
    )

    any_spec = pl.BlockSpec(memory_space=pl.ANY)

    pred, m_mask, d_mask, seg = pl.pallas_call(
        _gtnet_kernel,
        in_specs=[any_spec, any_spec, any_spec],
        out_specs=(any_spec, any_spec, any_spec, any_spec),
        out_shape=out_shape,
        scratch_shapes=[
            pltpu.VMEM((2, 2, _H, _W), jnp.float32),
            pltpu.VMEM((2, 1, _H, _W), jnp.int32),
            pltpu.VMEM((2, _IM_CH, _H, _W), jnp.float32),
            pltpu.VMEM((2, _IM_CH, _H, _W), jnp.float32),
            pltpu.VMEM((2, _N_CLASS, _H, _W), jnp.float32),
            pltpu.VMEM((2, _N_DEPTH, _H, _W), jnp.float32),
            pltpu.VMEM((2, 1, _H, _W), jnp.float32),
            pltpu.SemaphoreType.DMA((2,)),
            pltpu.SemaphoreType.DMA((2,)),
            pltpu.SemaphoreType.DMA((2,)),
            pltpu.SemaphoreType.DMA((2,)),
            pltpu.SemaphoreType.DMA((2,)),
            pltpu.SemaphoreType.DMA((2,)),
            pltpu.SemaphoreType.DMA((2,)),
        ],
        compiler_params=pltpu.CompilerParams(
            vmem_limit_bytes=48 * 1024 * 1024,
        ),
        name="gtnet_fused",
        interpret=interpret,
    )(gt_motion, dep, im_input)
    return pred, m_mask, d_mask, seg


# speedup vs baseline: 1.0276x; 1.0276x over previous
import jax
import jax.numpy as jnp
from jax.experimental import pallas as pl
from jax.experimental.pallas import tpu as pltpu

_M_RANGE = 2
_K = 5
_N_CLASS = 25
_N_DEPTH = 2
_IM_CH = 3
_EPS = 1e-8
_H = 256
_W = 256
_B = 8


def _shift_y(z, d):
    if d == 0:
        return z
    h, w = z.shape
    zpad = jnp.zeros((abs(d), w), z.dtype)
    if d > 0:
        return jnp.concatenate([z[d:, :], zpad], axis=0)
    return jnp.concatenate([zpad, z[:h + d, :]], axis=0)


def _shift_x(z, d):
    if d == 0:
        return z
    h, w = z.shape
    zpad = jnp.zeros((h, abs(d)), z.dtype)
    if d > 0:
        return jnp.concatenate([z[:, d:], zpad], axis=1)
    return jnp.concatenate([zpad, z[:, :w + d]], axis=1)


def _axis_weights(f_idx, frac):
    lo = 1.0 - frac
    w = []
    prev_hi = None
    for k in range(_K):
        e = f_idx == float(k)
        wk = jnp.where(e, lo, 0.0)
        if prev_hi is not None:
            wk = wk + prev_hi
        w.append(wk)
        if k + 1 < _K:
            prev_hi = jnp.where(e, frac, 0.0)
    return w


def _stencil(mot, dep, im_r, pred, mmask, dmask, seg):
    mx = mot[0]
    my = mot[1]
    fmx = jnp.floor(mx)
    fmy = jnp.floor(my)
    fx = mx - fmx
    fy = my - fmy
    ixf = fmx + float(_M_RANGE)
    iyf = fmy + float(_M_RANGE)

    d1 = (dep[0] == 1).astype(jnp.float32)
    dmask[0] = 1.0 - d1
    dmask[1] = d1

    im = [im_r[ch] for ch in range(_IM_CH)]

    wy = _axis_weights(iyf, fy)
    wx = _axis_weights(ixf, fx)

    sd1 = [_shift_y(d1, r - _M_RANGE) for r in range(_K)]
    sim = [[_shift_y(im[ch], r - _M_RANGE) for r in range(_K)]
           for ch in range(_IM_CH)]

    def acc(a, v):
        return v if a is None else a + v

    T = None
    S1 = None
    PT = [None] * _IM_CH
    P1 = [None] * _IM_CH

    for col in range(_K):
        wxc = wx[col]
        aT = None
        aS = None
        aPT = [None] * _IM_CH
        aP1 = [None] * _IM_CH
        for row in range(_K):
            t0 = wy[row] * wxc
            mmask[_K * row + col] = t0
            st0 = _shift_y(t0, row - _M_RANGE)
            u = st0 * sd1[row]
            aT = acc(aT, st0)
            aS = acc(aS, u)
            for ch in range(_IM_CH):
                aPT[ch] = acc(aPT[ch], st0 * sim[ch][row])
                aP1[ch] = acc(aP1[ch], u * sim[ch][row])
        dx = col - _M_RANGE
        T = acc(T, _shift_x(aT, dx))
        S1 = acc(S1, _shift_x(aS, dx))
        for ch in range(_IM_CH):
            PT[ch] = acc(PT[ch], _shift_x(aPT[ch], dx))
            P1[ch] = acc(P1[ch], _shift_x(aP1[ch], dx))

    eps_tot = float(_N_CLASS) * _EPS
    total1 = S1 + eps_tot
    total2 = (T - S1) + eps_tot
    ratio1 = 1.0 / total1
    f1 = 1.0 - jnp.maximum(1.0 - ratio1, 0.0)
    sum1 = total1 * f1
    left2 = jnp.maximum(1.0 - sum1, 0.0)
    ratio2 = left2 / total2
    f2 = 1.0 - jnp.maximum(1.0 - ratio2, 0.0)

    f12 = f1 - f2
    for ch in range(_IM_CH):
        pred[ch] = f12 * P1[ch] + f2 * PT[ch]
    seg[0] = 1.0 - (sum1 + total2 * f2)


def _gtnet_kernel(mot_hbm, dep_hbm, im_hbm,
                  pred_hbm, mmask_hbm, dmask_hbm, seg_hbm,
                  mot_buf, dep_buf, im_buf,
                  pred_buf, mmask_buf, dmask_buf, seg_buf,
                  mot_sem, dep_sem, im_sem,
                  pred_sem, mmask_sem, dmask_sem, seg_sem):

    def dma_in(slot, step):
        pltpu.make_async_copy(mot_hbm.at[step], mot_buf.at[slot],
                              mot_sem.at[slot]).start()
        pltpu.make_async_copy(dep_hbm.at[step], dep_buf.at[slot],
                              dep_sem.at[slot]).start()
        pltpu.make_async_copy(im_hbm.at[step, pl.ds(_IM_CH, _IM_CH)],
                              im_buf.at[slot], im_sem.at[slot]).start()

    def wait_in(slot):
        pltpu.make_async_copy(mot_hbm.at[0], mot_buf.at[slot],
                              mot_sem.at[slot]).wait()
        pltpu.make_async_copy(dep_hbm.at[0], dep_buf.at[slot],
                              dep_sem.at[slot]).wait()
        pltpu.make_async_copy(im_hbm.at[0, pl.ds(_IM_CH, _IM_CH)],
                              im_buf.at[slot], im_sem.at[slot]).wait()

    def dma_out(slot, step):
        pltpu.make_async_copy(pred_buf.at[slot], pred_hbm.at[step],
                              pred_sem.at[slot]).start()
        pltpu.make_async_copy(mmask_buf.at[slot], mmask_hbm.at[step],
                              mmask_sem.at[slot]).start()
        pltpu.make_async_copy(dmask_buf.at[slot], dmask_hbm.at[step],
                              dmask_sem.at[slot]).start()
        pltpu.make_async_copy(seg_buf.at[slot], seg_hbm.at[step],
                              seg_sem.at[slot]).start()

    def wait_out(slot):
        pltpu.make_async_copy(pred_buf.at[slot], pred_hbm.at[0],
                              pred_sem.at[slot]).wait()
        pltpu.make_async_copy(mmask_buf.at[slot], mmask_hbm.at[0],
                              mmask_sem.at[slot]).wait()
        pltpu.make_async_copy(dmask_buf.at[slot], dmask_hbm.at[0],
                              dmask_sem.at[slot]).wait()
        pltpu.make_async_copy(seg_buf.at[slot], seg_hbm.at[0],
                              seg_sem.at[slot]).wait()

    dma_in(0, 0)

    def body(step, _):
        cur = jax.lax.rem(step, 2)
        nxt = jax.lax.rem(step + 1, 2)

        @pl.when(step + 1 < _B)
        def _():
            dma_in(nxt, step + 1)

        wait_in(cur)

        @pl.when(step >= 2)
        def _():
            wait_out(cur)

        _stencil(mot_buf.at[cur], dep_buf.at[cur], im_buf.at[cur],
                 pred_buf.at[cur], mmask_buf.at[cur], dmask_buf.at[cur],
                 seg_buf.at[cur])
        dma_out(cur, step)
        return ()

    jax.lax.fori_loop(0, _B, body, ())
    wait_out(jax.lax.rem(_B - 2, 2))
    wait_out(jax.lax.rem(_B - 1, 2))


def kernel(im_input, gt_motion, gt_depth, m_kernel, *, interpret=False):
    B = gt_motion.shape[0]
    dep = gt_depth.astype(jnp.int32)

    out_shape = (
        jax.ShapeDtypeStruct((B, _IM_CH, _H, _W), jnp.float32),
        jax.ShapeDtypeStruct((B, _N_CLASS, _H, _W), jnp.float32),
        jax.ShapeDtypeStruct((B, _N_DEPTH, _H, _W), jnp.float32),
        jax.ShapeDtypeStruct((B, 1, _H, _W), jnp.float32),
    )

    any_spec = pl.BlockSpec(memory_space=pl.ANY)

    pred, m_mask, d_mask, seg = pl.pallas_call(
        _gtnet_kernel,
        in_specs=[any_spec, any_spec, any_spec],
        out_specs=(any_spec, any_spec, any_spec, any_spec),
        out_shape=out_shape,
        scratch_shapes=[
            pltpu.VMEM((2, 2, _H, _W), jnp.float32),
            pltpu.VMEM((2, 1, _H, _W), jnp.int32),
            pltpu.VMEM((2, _IM_CH, _H, _W), jnp.float32),
            pltpu.VMEM((2, _IM_CH, _H, _W), jnp.float32),
            pltpu.VMEM((2, _N_CLASS, _H, _W), jnp.float32),
            pltpu.VMEM((2, _N_DEPTH, _H, _W), jnp.float32),
            pltpu.VMEM((2, 1, _H, _W), jnp.float32),
            pltpu.SemaphoreType.DMA((2,)),
            pltpu.SemaphoreType.DMA((2,)),
            pltpu.SemaphoreType.DMA((2,)),
            pltpu.SemaphoreType.DMA((2,)),
            pltpu.SemaphoreType.DMA((2,)),
            pltpu.SemaphoreType.DMA((2,)),
            pltpu.SemaphoreType.DMA((2,)),
        ],
        compiler_params=pltpu.CompilerParams(
            vmem_limit_bytes=48 * 1024 * 1024,
        ),
        name="gtnet_fused",
        interpret=interpret,
    )(gt_motion, dep, im_input)
    return pred, m_mask, d_mask, seg
